# baseline (speedup 1.0000x reference)
import functools

import jax
import jax.numpy as jnp
from jax import lax
from jax.experimental import pallas as pl
from jax.experimental.pallas import tpu as pltpu
from jax.experimental.pallas import tpu_sc as plsc

VOCAB_SIZE = 50257
DIM = 1024
CONTEXT_LENGTH = 2048
BATCH = 4
SEQ_LEN = 2048

_NC = 2
_NS = 16
_NW = _NC * _NS
_BS = BATCH * SEQ_LEN
_PER_W = _BS // _NW
_C = 32
_NCHUNKS = _PER_W // _C
_LANES = 16
_GROUPS = DIM // _LANES


def _make_kernel():
  mesh = plsc.VectorSubcoreMesh(core_axis_name="c", subcore_axis_name="s")

  @functools.partial(
      pl.kernel,
      out_type=jax.ShapeDtypeStruct((_BS, DIM), jnp.float32),
      mesh=mesh,
      scratch_types=[
          pltpu.VMEM((_C,), jnp.int32),
          pltpu.VMEM((_C, DIM), jnp.float32),
          pltpu.VMEM((_C, DIM), jnp.float32),
          pltpu.SemaphoreType.DMA,
      ],
  )
  def k(idx_hbm, tok_hbm, pos_hbm, out_hbm, idx_v, tok_v, pos_v, sem):
    wid = lax.axis_index("s") * _NC + lax.axis_index("c")
    base = wid * _PER_W

    @pl.loop(0, _NCHUNKS)
    def chunk_body(c):
      off = base + c * _C
      s0 = lax.rem(off, SEQ_LEN)
      pltpu.sync_copy(idx_hbm.at[pl.ds(off, _C)], idx_v)
      gather = pltpu.async_copy(tok_hbm.at[idx_v], tok_v, sem)
      pltpu.sync_copy(pos_hbm.at[pl.ds(s0, _C)], pos_v)
      gather.wait()

      @pl.loop(0, _C)
      def add_row(i):
        for j in range(_GROUPS):
          sl = pl.ds(j * _LANES, _LANES)
          tok_v[i, sl] = tok_v[i, sl] + pos_v[i, sl]

      pltpu.sync_copy(tok_v, out_hbm.at[pl.ds(off, _C)])

  return k


_kernel_fn = _make_kernel()


def kernel(in_idx, tok_emb, pos_emb):
  idx_flat = in_idx.reshape(_BS).astype(jnp.int32)
  out = _kernel_fn(idx_flat, tok_emb, pos_emb)
  return out.reshape(BATCH, SEQ_LEN, DIM)


# speedup vs baseline: 1.2539x; 1.2539x over previous
import functools

import jax
import jax.numpy as jnp
from jax import lax
from jax.experimental import pallas as pl
from jax.experimental.pallas import tpu as pltpu
from jax.experimental.pallas import tpu_sc as plsc

VOCAB_SIZE = 50257
DIM = 1024
CONTEXT_LENGTH = 2048
BATCH = 4
SEQ_LEN = 2048

_NC = 2
_NS = 16
_NW = _NC * _NS
_BS = BATCH * SEQ_LEN
_PER_W = _BS // _NW
_C = 16
_NCHUNKS = _PER_W // _C
_NBUF = 2
_LANES = 16
_GROUPS = DIM // _LANES


def _make_kernel():
  mesh = plsc.VectorSubcoreMesh(core_axis_name="c", subcore_axis_name="s")

  @functools.partial(
      pl.kernel,
      out_type=jax.ShapeDtypeStruct((_BS, DIM), jnp.float32),
      mesh=mesh,
      scratch_types=[
          pltpu.VMEM((_PER_W,), jnp.int32),
          pltpu.VMEM((_NBUF, _C, DIM), jnp.float32),
          pltpu.VMEM((_NBUF, _C, DIM), jnp.float32),
          pltpu.VMEM((_NBUF, _C, DIM), jnp.float32),
          pltpu.SemaphoreType.DMA((_NBUF,)),
          pltpu.SemaphoreType.DMA((_NBUF,)),
          pltpu.SemaphoreType.DMA((_NBUF,)),
      ],
  )
  def k(idx_hbm, tok_hbm, pos_hbm, out_hbm,
        idx_all, tok_v, pos_v, out_v, gsem, psem, osem):
    wid = lax.axis_index("s") * _NC + lax.axis_index("c")
    base = wid * _PER_W

    pltpu.sync_copy(idx_hbm.at[pl.ds(base, _PER_W)], idx_all)

    def start_fetch(c, b):
      off = base + c * _C
      s0 = lax.rem(off, SEQ_LEN)
      pltpu.async_copy(
          tok_hbm.at[idx_all.at[pl.ds(c * _C, _C)]], tok_v.at[b], gsem.at[b])
      pltpu.async_copy(pos_hbm.at[pl.ds(s0, _C)], pos_v.at[b], psem.at[b])

    for b in range(_NBUF):
      start_fetch(b, b)

    @pl.loop(0, _NCHUNKS, step=_NBUF)
    def body(k0):
      for b in range(_NBUF):
        c = k0 + b
        pltpu.make_async_copy(
            tok_hbm.at[idx_all.at[pl.ds(0, _C)]], tok_v.at[b], gsem.at[b]
        ).wait()
        pltpu.make_async_copy(
            pos_hbm.at[pl.ds(0, _C)], pos_v.at[b], psem.at[b]).wait()

        @pl.when(c >= _NBUF)
        def _():
          pltpu.make_async_copy(
              out_v.at[b], out_hbm.at[pl.ds(0, _C)], osem.at[b]).wait()

        @pl.loop(0, _C)
        def add_row(i):
          for j in range(_GROUPS):
            sl = pl.ds(j * _LANES, _LANES)
            out_v[b, i, sl] = tok_v[b, i, sl] + pos_v[b, i, sl]

        off = base + c * _C
        pltpu.async_copy(out_v.at[b], out_hbm.at[pl.ds(off, _C)], osem.at[b])

        @pl.when(c + _NBUF < _NCHUNKS)
        def _():
          start_fetch(c + _NBUF, b)

    for b in range(_NBUF):
      pltpu.make_async_copy(
          out_v.at[b], out_hbm.at[pl.ds(0, _C)], osem.at[b]).wait()

  return k


_kernel_fn = _make_kernel()


def kernel(in_idx, tok_emb, pos_emb):
  idx_flat = in_idx.reshape(_BS).astype(jnp.int32)
  out = _kernel_fn(idx_flat, tok_emb, pos_emb)
  return out.reshape(BATCH, SEQ_LEN, DIM)


# speedup vs baseline: 1.2981x; 1.0353x over previous
import functools

import jax
import jax.numpy as jnp
from jax import lax
from jax.experimental import pallas as pl
from jax.experimental.pallas import tpu as pltpu
from jax.experimental.pallas import tpu_sc as plsc

VOCAB_SIZE = 50257
DIM = 1024
CONTEXT_LENGTH = 2048
BATCH = 4
SEQ_LEN = 2048

_NC = 2
_NS = 16
_NW = _NC * _NS
_BS = BATCH * SEQ_LEN
_SW = SEQ_LEN // _NW
_C = 16
_SCHUNKS = _SW // _C
_NCHUNKS = _SCHUNKS * BATCH
_NBUF = 4
_PBUF = 2
_LOOKAHEAD = 2
_LANES = 16
_GROUPS = DIM // _LANES


def _make_kernel():
  mesh = plsc.VectorSubcoreMesh(core_axis_name="c", subcore_axis_name="s")

  @functools.partial(
      pl.kernel,
      out_type=jax.ShapeDtypeStruct((_BS, DIM), jnp.float32),
      mesh=mesh,
      scratch_types=[
          pltpu.VMEM((BATCH * _SW,), jnp.int32),
          pltpu.VMEM((_NBUF, _C, DIM), jnp.float32),
          pltpu.VMEM((_PBUF, _C, DIM), jnp.float32),
          pltpu.SemaphoreType.DMA((_NBUF,)),
          pltpu.SemaphoreType.DMA((_PBUF,)),
          pltpu.SemaphoreType.DMA((_NBUF,)),
      ],
  )
  def k(idx_hbm, tok_hbm, pos_hbm, out_hbm,
        idx_all, tok_v, pos_v, gsem, psem, osem):
    wid = lax.axis_index("s") * _NC + lax.axis_index("c")
    s0 = wid * _SW

    for b in range(BATCH):
      pltpu.sync_copy(idx_hbm.at[pl.ds(b * SEQ_LEN + s0, _SW)],
                      idx_all.at[pl.ds(b * _SW, _SW)])

    def start_gather(c, buf):
      sc, b = divmod(c, BATCH)
      pltpu.async_copy(
          tok_hbm.at[idx_all.at[pl.ds(b * _SW + sc * _C, _C)]],
          tok_v.at[buf], gsem.at[buf])

    def start_pos(sc):
      pltpu.async_copy(pos_hbm.at[pl.ds(s0 + sc * _C, _C)],
                       pos_v.at[sc % _PBUF], psem.at[sc % _PBUF])

    start_pos(0)
    for c in range(_LOOKAHEAD):
      start_gather(c, c)

    for c in range(_NCHUNKS):
      sc, b = divmod(c, BATCH)
      u = c % _NBUF

      if c + _LOOKAHEAD < _NCHUNKS:
        bg = (u + _LOOKAHEAD) % _NBUF
        if c >= _LOOKAHEAD:
          pltpu.make_async_copy(
              tok_v.at[bg], out_hbm.at[pl.ds(0, _C)], osem.at[bg]).wait()
        start_gather(c + _LOOKAHEAD, bg)

      if b == 0:
        if sc + 1 < _SCHUNKS:
          start_pos(sc + 1)
        pltpu.make_async_copy(
            pos_hbm.at[pl.ds(0, _C)], pos_v.at[sc % _PBUF],
            psem.at[sc % _PBUF]).wait()

      pltpu.make_async_copy(
          tok_hbm.at[idx_all.at[pl.ds(0, _C)]], tok_v.at[u], gsem.at[u]
      ).wait()

      @pl.loop(0, _C)
      def add_row(i):
        for j in range(_GROUPS):
          sl = pl.ds(j * _LANES, _LANES)
          plsc.addupdate(tok_v.at[u, i, sl], pos_v[sc % _PBUF, i, sl])

      off = b * SEQ_LEN + s0 + sc * _C
      pltpu.async_copy(tok_v.at[u], out_hbm.at[pl.ds(off, _C)], osem.at[u])

    for u in range(_NBUF):
      pltpu.make_async_copy(
          tok_v.at[u], out_hbm.at[pl.ds(0, _C)], osem.at[u]).wait()

  return k


_kernel_fn = _make_kernel()


def kernel(in_idx, tok_emb, pos_emb):
  idx_flat = in_idx.reshape(_BS).astype(jnp.int32)
  out = _kernel_fn(idx_flat, tok_emb, pos_emb)
  return out.reshape(BATCH, SEQ_LEN, DIM)


# speedup vs baseline: 1.3427x; 1.0344x over previous
import functools

import jax
import jax.numpy as jnp
from jax import lax
from jax.experimental import pallas as pl
from jax.experimental.pallas import tpu as pltpu
from jax.experimental.pallas import tpu_sc as plsc

VOCAB_SIZE = 50257
DIM = 1024
CONTEXT_LENGTH = 2048
BATCH = 4
SEQ_LEN = 2048

_NC = 2
_NS = 16
_NW = _NC * _NS
_BS = BATCH * SEQ_LEN
_SW = SEQ_LEN // _NW
_C = 16
_SCHUNKS = _SW // _C
_NG = _SCHUNKS * BATCH
_NBUF = 5
_PBUF = 2
_LANES = 16
_GROUPS = DIM // _LANES


def _make_kernel():
  mesh = plsc.VectorSubcoreMesh(core_axis_name="c", subcore_axis_name="s")

  @functools.partial(
      pl.kernel,
      out_type=jax.ShapeDtypeStruct((_BS, DIM), jnp.float32),
      mesh=mesh,
      scratch_types=[
          pltpu.VMEM((BATCH * _SW,), jnp.int32),
          pltpu.VMEM((_NBUF, _C, DIM), jnp.float32),
          pltpu.VMEM((_PBUF, _C, DIM), jnp.float32),
          pltpu.SemaphoreType.DMA((_NBUF,)),
          pltpu.SemaphoreType.DMA((_PBUF,)),
          pltpu.SemaphoreType.DMA((_NBUF,)),
      ],
  )
  def k(idx_hbm, tok_hbm, pos_hbm, out_hbm,
        idx_all, tok_v, pos_v, gsem, psem, osem):
    wid = lax.axis_index("s") * _NC + lax.axis_index("c")
    s0 = wid * _SW

    for b in range(BATCH):
      pltpu.sync_copy(idx_hbm.at[b, pl.ds(s0, _SW)],
                      idx_all.at[pl.ds(b * _SW, _SW)])

    def start_gather(g):
      sc, b = divmod(g, BATCH)
      pltpu.async_copy(
          tok_hbm.at[idx_all.at[pl.ds(b * _SW + sc * _C, _C)]],
          tok_v.at[g % _NBUF], gsem.at[g % _NBUF])

    def start_pos(sc):
      pltpu.async_copy(pos_hbm.at[pl.ds(s0 + sc * _C, _C)],
                       pos_v.at[sc % _PBUF], psem.at[sc % _PBUF])

    def wait_store(slot):
      pltpu.make_async_copy(
          tok_v.at[slot], out_hbm.at[pl.ds(0, _C)], osem.at[slot]).wait()

    start_pos(0)
    start_pos(1)
    for g in range(_NBUF):
      start_gather(g)

    for sc in range(_SCHUNKS):
      slots = [(sc * BATCH + b) % _NBUF for b in range(BATCH)]

      for t in slots:
        pltpu.make_async_copy(
            tok_hbm.at[idx_all.at[pl.ds(0, _C)]], tok_v.at[t], gsem.at[t]
        ).wait()
      pltpu.make_async_copy(
          pos_hbm.at[pl.ds(0, _C)], pos_v.at[sc % _PBUF],
          psem.at[sc % _PBUF]).wait()

      @pl.loop(0, _C)
      def add_row(i):
        for j in range(_GROUPS):
          sl = pl.ds(j * _LANES, _LANES)
          p = pos_v[sc % _PBUF, i, sl]
          for t in slots:
            plsc.addupdate(tok_v.at[t, i, sl], p)

      if sc + 2 < _SCHUNKS:
        start_pos(sc + 2)

      for b in range(BATCH):
        t = slots[b]
        off = b * SEQ_LEN + s0 + sc * _C
        pltpu.async_copy(tok_v.at[t], out_hbm.at[pl.ds(off, _C)], osem.at[t])

      for g in range(sc * BATCH + _NBUF, min((sc + 1) * BATCH + _NBUF, _NG)):
        wait_store(g % _NBUF)
        start_gather(g)

    for t in range(_NBUF):
      wait_store(t)

  return k


_kernel_fn = _make_kernel()


def kernel(in_idx, tok_emb, pos_emb):
  out = _kernel_fn(in_idx.astype(jnp.int32), tok_emb, pos_emb)
  return out.reshape(BATCH, SEQ_LEN, DIM)
